# baseline (speedup 1.0000x reference)
import functools

import jax
import jax.numpy as jnp
from jax import lax
from jax.experimental import pallas as pl
from jax.experimental.pallas import tpu as pltpu
from jax.experimental.pallas import tpu_sc as plsc

NE = 8
NK = 2
BLK = 256
NEG = -1e30
NW = 32


def _route_bin_body(x_ref, wg_ref, invt_ref, bias_ref,
                    w0_ref, w1_ref, pos_ref, meta_ref):
    f32 = jnp.float32
    T = x_ref.shape[0]
    TK = T * NK
    C = 128

    g = jnp.dot(x_ref[...], wg_ref[...], preferred_element_type=f32)
    g = g * invt_ref[...] + bias_ref[...]
    i1 = jnp.argmax(g, axis=1).astype(jnp.int32)
    l1 = jnp.max(g, axis=1)
    lanes = lax.broadcasted_iota(jnp.int32, g.shape, 1)
    g2 = jnp.where(lanes == i1[:, None], NEG, g)
    i2 = jnp.argmax(g2, axis=1).astype(jnp.int32)
    l2 = jnp.max(g2, axis=1)
    wa = 1.0 / (1.0 + jnp.exp(l2 - l1))
    w0_ref[...] = jnp.broadcast_to(wa[:, None], (T, 16))
    w1_ref[...] = jnp.broadcast_to((1.0 - wa)[:, None], (T, 16))

    M = jnp.concatenate([(lanes == i1[:, None]).astype(f32),
                         (lanes == i2[:, None]).astype(f32)], axis=0)

    li = lax.broadcasted_iota(jnp.int32, (C, C), 0)
    lj = lax.broadcasted_iota(jnp.int32, (C, C), 1)
    ltri = (lj <= li).astype(jnp.bfloat16)
    ones = jnp.ones((C, C), jnp.bfloat16)

    counts = jnp.sum(M, axis=0, keepdims=True)
    nblk = jnp.floor((counts + (BLK - 1)) * (1.0 / BLK))
    utri = (li < lj).astype(jnp.bfloat16)
    bstart = jnp.dot(nblk.astype(jnp.bfloat16), utri,
                     preferred_element_type=f32)
    start = bstart * BLK
    used = jnp.sum(nblk, axis=1, keepdims=True)

    run = jnp.zeros((1, C), f32)
    for c in range(TK // C):
        Mc = M[c * C:(c + 1) * C, :]
        cumc = jnp.dot(ltri, Mc.astype(jnp.bfloat16),
                       preferred_element_type=f32) + run
        posc = jnp.sum(Mc * (cumc - 1.0 + start), axis=1, keepdims=True)
        pos_ref[c * C:(c + 1) * C, :] = posc.astype(jnp.int32)
        run = run + jnp.sum(Mc, axis=0, keepdims=True)

    eq = (li == lj).astype(f32)
    bstart_s = jnp.dot((eq * bstart).astype(jnp.bfloat16), ones,
                       preferred_element_type=f32)
    nblk_s = jnp.dot((eq * nblk).astype(jnp.bfloat16), ones,
                     preferred_element_type=f32)
    bidx = lj.astype(f32)
    inr = (bidx >= bstart_s) & (bidx < bstart_s + nblk_s)
    bexp_real = jnp.sum(jnp.where(inr, li.astype(f32), 0.0), axis=0,
                        keepdims=True)
    lane1 = lanes[:1, :].astype(f32)
    elast = jnp.max(jnp.where(counts > 0, lane1, -1.0), axis=1,
                    keepdims=True)
    bvalid = lane1 < used
    bexp = jnp.where(bvalid, bexp_real, elast)
    xi = jnp.where(bvalid, lane1, used - 1.0)
    meta = jnp.concatenate(
        [bexp, xi, bvalid.astype(f32), jnp.zeros((5, C), f32)], axis=0)
    meta_ref[...] = meta.astype(jnp.int32)


def _route_bin(x, wg_pad, invt_pad, bias_pad):
    T, D = x.shape
    TK = T * NK
    return pl.pallas_call(
        _route_bin_body,
        in_specs=[
            pl.BlockSpec((T, D), lambda: (0, 0)),
            pl.BlockSpec((D, 128), lambda: (0, 0)),
            pl.BlockSpec((1, 128), lambda: (0, 0)),
            pl.BlockSpec((1, 128), lambda: (0, 0)),
        ],
        out_specs=[
            pl.BlockSpec((T, 16), lambda: (0, 0)),
            pl.BlockSpec((T, 16), lambda: (0, 0)),
            pl.BlockSpec((TK, 1), lambda: (0, 0)),
            pl.BlockSpec((8, 128), lambda: (0, 0)),
        ],
        out_shape=[
            jax.ShapeDtypeStruct((T, 16), jnp.float32),
            jax.ShapeDtypeStruct((T, 16), jnp.float32),
            jax.ShapeDtypeStruct((TK, 1), jnp.int32),
            jax.ShapeDtypeStruct((8, 128), jnp.int32),
        ],
    )(x, wg_pad, invt_pad, bias_pad)


def _sc_scatter_x(x, pos3, npad):
    T, D = x.shape
    jc = pos3.shape[1]
    cb = pos3.shape[2]
    mesh = plsc.VectorSubcoreMesh(core_axis_name="c", subcore_axis_name="s")

    @functools.partial(
        pl.kernel, mesh=mesh,
        out_type=jax.ShapeDtypeStruct((npad, D), jnp.float32),
        scratch_types=[
            pltpu.VMEM((jc, cb), jnp.int32),
            pltpu.VMEM((cb, D), jnp.float32),
            pltpu.SemaphoreType.DMA,
        ],
    )
    def k(x_hbm, pos_hbm, xs_hbm, pos_v, rows_v, sem):
        wid = lax.axis_index("s") * 2 + lax.axis_index("c")
        pltpu.sync_copy(pos_hbm.at[wid], pos_v)
        tok_base = wid * (jc * cb) - jnp.where(wid >= NW // 2, T, 0)
        for c in range(jc):
            pltpu.sync_copy(x_hbm.at[pl.ds(tok_base + c * cb, cb)], rows_v)
            pltpu.async_copy(rows_v, xs_hbm.at[pos_v.at[c]], sem).wait()

    return k(x, pos3)


def _ffn_body(meta_ref, xs_ref, w1_ref, w3_ref, w2_ref, ys_ref):
    i = pl.program_id(0)
    f = pl.program_id(1)

    @pl.when(meta_ref[2, i] != 0)
    def _():
        xb = xs_ref[...].astype(jnp.bfloat16)
        a = jnp.dot(xb, w1_ref[0].astype(jnp.bfloat16),
                    preferred_element_type=jnp.float32)
        b = jnp.dot(xb, w3_ref[0].astype(jnp.bfloat16),
                    preferred_element_type=jnp.float32)
        h = (a * jax.nn.sigmoid(a) * b).astype(jnp.bfloat16)
        y = jnp.dot(h, w2_ref[0].astype(jnp.bfloat16),
                    preferred_element_type=jnp.float32)

        @pl.when(f == 0)
        def _():
            ys_ref[...] = y

        @pl.when(f != 0)
        def _():
            ys_ref[...] += y


def _ffn(x_sorted, w1, w3, w2, meta, nblk):
    D = x_sorted.shape[1]
    F = w1.shape[2]
    NF = 2
    FC = F // NF
    grid_spec = pltpu.PrefetchScalarGridSpec(
        num_scalar_prefetch=1,
        grid=(nblk, NF),
        in_specs=[
            pl.BlockSpec((BLK, D), lambda i, f, m: (m[1, i], 0)),
            pl.BlockSpec((1, D, FC), lambda i, f, m: (m[0, i], 0, f)),
            pl.BlockSpec((1, D, FC), lambda i, f, m: (m[0, i], 0, f)),
            pl.BlockSpec((1, FC, D), lambda i, f, m: (m[0, i], f, 0)),
        ],
        out_specs=pl.BlockSpec((BLK, D), lambda i, f, m: (m[1, i], 0)),
    )
    return pl.pallas_call(
        _ffn_body,
        grid_spec=grid_spec,
        out_shape=jax.ShapeDtypeStruct(x_sorted.shape, jnp.float32),
    )(meta, x_sorted, w1, w3, w2)


def _sc_combine(ys, pos2, w0f, w1f, T):
    npad, D = ys.shape
    per = T // NW
    GC = 16
    mesh = plsc.VectorSubcoreMesh(core_axis_name="c", subcore_axis_name="s")

    @functools.partial(
        pl.kernel, mesh=mesh,
        out_type=jax.ShapeDtypeStruct((T, D), jnp.float32),
        scratch_types=[
            pltpu.VMEM((per,), jnp.int32),
            pltpu.VMEM((per,), jnp.int32),
            pltpu.VMEM((per * 16,), jnp.float32),
            pltpu.VMEM((per * 16,), jnp.float32),
            pltpu.VMEM((GC, D), jnp.float32),
            pltpu.VMEM((GC, D), jnp.float32),
            pltpu.VMEM((GC, D), jnp.float32),
            pltpu.SemaphoreType.DMA,
            pltpu.SemaphoreType.DMA,
        ],
    )
    def k(ys_hbm, pos_hbm, w0_hbm, w1_hbm, out_hbm,
          idx0_v, idx1_v, w0_v, w1_v, rows0_v, rows1_v, out_v, sem0, sem1):
        wid = lax.axis_index("s") * 2 + lax.axis_index("c")
        tbase = wid * per
        pltpu.sync_copy(pos_hbm.at[0, wid], idx0_v)
        pltpu.sync_copy(pos_hbm.at[1, wid], idx1_v)
        pltpu.sync_copy(w0_hbm.at[pl.ds(tbase * 16, per * 16)], w0_v)
        pltpu.sync_copy(w1_hbm.at[pl.ds(tbase * 16, per * 16)], w1_v)

        def body(c, carry):
            cp0 = pltpu.async_copy(
                ys_hbm.at[idx0_v.at[pl.ds(c * GC, GC)]], rows0_v, sem0)
            cp1 = pltpu.async_copy(
                ys_hbm.at[idx1_v.at[pl.ds(c * GC, GC)]], rows1_v, sem1)
            cp0.wait()
            cp1.wait()
            for t in range(GC):
                wv0 = w0_v[pl.ds(c * (GC * 16) + t * 16, 16)]
                wv1 = w1_v[pl.ds(c * (GC * 16) + t * 16, 16)]
                for j in range(D // 16):
                    sl = pl.ds(j * 16, 16)
                    out_v[t, sl] = (wv0 * rows0_v[t, sl] +
                                    wv1 * rows1_v[t, sl])
            pltpu.sync_copy(out_v, out_hbm.at[pl.ds(tbase + c * GC, GC)])
            return carry

        lax.fori_loop(0, per // GC, body, 0)

    return k(ys, pos2, w0f, w1f)


def kernel(x, pressure, temperature, Wg, w1, w3, w2):
    T, D = x.shape
    E = Wg.shape[1]
    TK = T * NK
    NBLK = TK // BLK + NE
    NPAD = NBLK * BLK

    invt = 1.0 / jnp.clip(temperature, 0.3, 3.0)
    bias = 0.1 * jnp.clip(pressure, -1.0, 1.0)
    invt_pad = jnp.zeros((1, 128), jnp.float32).at[0, :E].set(invt)
    bias_pad = jnp.full((1, 128), NEG, jnp.float32).at[0, :E].set(bias)
    wg_pad = jnp.zeros((D, 128), jnp.float32).at[:, :E].set(Wg)

    w0b, w1b, pos, meta = _route_bin(x, wg_pad, invt_pad, bias_pad)

    x_sorted = _sc_scatter_x(x, pos.reshape(NW, TK // NW // 64, 64), NPAD)

    y_sorted = _ffn(x_sorted, w1, w3, w2, meta, NBLK)

    return _sc_combine(y_sorted, pos.reshape(NK, NW, T // NW),
                       w0b.reshape(T * 16), w1b.reshape(T * 16), T)


# speedup vs baseline: 1.2983x; 1.2983x over previous
import functools

import jax
import jax.numpy as jnp
from jax import lax
from jax.experimental import pallas as pl
from jax.experimental.pallas import tpu as pltpu
from jax.experimental.pallas import tpu_sc as plsc

NE = 8
NK = 2
BLK = 128
NEG = -1e30
NW = 32


def _route_bin_body(x_ref, wg_ref, invt_ref, bias_ref,
                    w0_ref, w1_ref, pos_ref, meta_ref):
    f32 = jnp.float32
    T = x_ref.shape[0]
    TK = T * NK
    C = 128

    g = jnp.dot(x_ref[...], wg_ref[...], preferred_element_type=f32)
    g = g * invt_ref[...] + bias_ref[...]
    i1 = jnp.argmax(g, axis=1).astype(jnp.int32)
    l1 = jnp.max(g, axis=1)
    lanes = lax.broadcasted_iota(jnp.int32, g.shape, 1)
    g2 = jnp.where(lanes == i1[:, None], NEG, g)
    i2 = jnp.argmax(g2, axis=1).astype(jnp.int32)
    l2 = jnp.max(g2, axis=1)
    wa = 1.0 / (1.0 + jnp.exp(l2 - l1))
    w0_ref[...] = jnp.broadcast_to(wa[:, None], (T, 16))
    w1_ref[...] = jnp.broadcast_to((1.0 - wa)[:, None], (T, 16))

    M = jnp.concatenate([(lanes == i1[:, None]).astype(f32),
                         (lanes == i2[:, None]).astype(f32)], axis=0)

    li = lax.broadcasted_iota(jnp.int32, (C, C), 0)
    lj = lax.broadcasted_iota(jnp.int32, (C, C), 1)
    ltri = (lj <= li).astype(jnp.bfloat16)
    ones = jnp.ones((C, C), jnp.bfloat16)

    counts = jnp.sum(M, axis=0, keepdims=True)
    nblk = jnp.floor((counts + (BLK - 1)) * (1.0 / BLK))
    utri = (li < lj).astype(jnp.bfloat16)
    bstart = jnp.dot(nblk.astype(jnp.bfloat16), utri,
                     preferred_element_type=f32)
    start = bstart * BLK
    used = jnp.sum(nblk, axis=1, keepdims=True)

    run = jnp.zeros((1, C), f32)
    for c in range(TK // C):
        Mc = M[c * C:(c + 1) * C, :]
        cumc = jnp.dot(ltri, Mc.astype(jnp.bfloat16),
                       preferred_element_type=f32) + run
        posc = jnp.sum(Mc * (cumc - 1.0 + start), axis=1, keepdims=True)
        pos_ref[c * C:(c + 1) * C, :] = posc.astype(jnp.int32)
        run = run + jnp.sum(Mc, axis=0, keepdims=True)

    eq = (li == lj).astype(f32)
    bstart_s = jnp.dot((eq * bstart).astype(jnp.bfloat16), ones,
                       preferred_element_type=f32)
    nblk_s = jnp.dot((eq * nblk).astype(jnp.bfloat16), ones,
                     preferred_element_type=f32)
    bidx = lj.astype(f32)
    inr = (bidx >= bstart_s) & (bidx < bstart_s + nblk_s)
    bexp_real = jnp.sum(jnp.where(inr, li.astype(f32), 0.0), axis=0,
                        keepdims=True)
    lane1 = lanes[:1, :].astype(f32)
    elast = jnp.max(jnp.where(counts > 0, lane1, -1.0), axis=1,
                    keepdims=True)
    bvalid = lane1 < used
    bexp = jnp.where(bvalid, bexp_real, elast)
    xi = jnp.where(bvalid, lane1, used - 1.0)
    meta = jnp.concatenate(
        [bexp, xi, bvalid.astype(f32), jnp.zeros((5, C), f32)], axis=0)
    meta_ref[...] = meta.astype(jnp.int32)


def _route_bin(x, wg_pad, invt_pad, bias_pad):
    T, D = x.shape
    TK = T * NK
    return pl.pallas_call(
        _route_bin_body,
        in_specs=[
            pl.BlockSpec((T, D), lambda: (0, 0)),
            pl.BlockSpec((D, 128), lambda: (0, 0)),
            pl.BlockSpec((1, 128), lambda: (0, 0)),
            pl.BlockSpec((1, 128), lambda: (0, 0)),
        ],
        out_specs=[
            pl.BlockSpec((T, 16), lambda: (0, 0)),
            pl.BlockSpec((T, 16), lambda: (0, 0)),
            pl.BlockSpec((TK, 1), lambda: (0, 0)),
            pl.BlockSpec((8, 128), lambda: (0, 0)),
        ],
        out_shape=[
            jax.ShapeDtypeStruct((T, 16), jnp.float32),
            jax.ShapeDtypeStruct((T, 16), jnp.float32),
            jax.ShapeDtypeStruct((TK, 1), jnp.int32),
            jax.ShapeDtypeStruct((8, 128), jnp.int32),
        ],
    )(x, wg_pad, invt_pad, bias_pad)


def _sc_scatter_x(x, pos3, npad):
    T, D = x.shape
    jc = pos3.shape[1]
    cb = pos3.shape[2]
    mesh = plsc.VectorSubcoreMesh(core_axis_name="c", subcore_axis_name="s")

    @functools.partial(
        pl.kernel, mesh=mesh,
        out_type=jax.ShapeDtypeStruct((npad, D), jnp.float32),
        scratch_types=[
            pltpu.VMEM((jc, cb), jnp.int32),
            pltpu.VMEM((cb, D), jnp.float32),
            pltpu.SemaphoreType.DMA,
        ],
    )
    def k(x_hbm, pos_hbm, xs_hbm, pos_v, rows_v, sem):
        wid = lax.axis_index("s") * 2 + lax.axis_index("c")
        pltpu.sync_copy(pos_hbm.at[wid], pos_v)
        tok_base = wid * (jc * cb) - jnp.where(wid >= NW // 2, T, 0)
        for c in range(jc):
            pltpu.sync_copy(x_hbm.at[pl.ds(tok_base + c * cb, cb)], rows_v)
            pltpu.async_copy(rows_v, xs_hbm.at[pos_v.at[c]], sem).wait()

    return k(x, pos3)


def _ffn_body(meta_ref, xs_ref, w1_ref, w3_ref, w2_ref, ys_ref):
    i = pl.program_id(0)

    @pl.when(meta_ref[2, i] != 0)
    def _():
        xb = xs_ref[...].astype(jnp.bfloat16)
        a = jnp.dot(xb, w1_ref[0].astype(jnp.bfloat16),
                    preferred_element_type=jnp.float32)
        b = jnp.dot(xb, w3_ref[0].astype(jnp.bfloat16),
                    preferred_element_type=jnp.float32)
        h = (a * jax.nn.sigmoid(a) * b).astype(jnp.bfloat16)
        ys_ref[...] = jnp.dot(h, w2_ref[0].astype(jnp.bfloat16),
                              preferred_element_type=jnp.float32)


def _ffn(x_sorted, w1, w3, w2, meta, nblk):
    D = x_sorted.shape[1]
    F = w1.shape[2]
    grid_spec = pltpu.PrefetchScalarGridSpec(
        num_scalar_prefetch=1,
        grid=(nblk,),
        in_specs=[
            pl.BlockSpec((BLK, D), lambda i, m: (m[1, i], 0)),
            pl.BlockSpec((1, D, F), lambda i, m: (m[0, i], 0, 0)),
            pl.BlockSpec((1, D, F), lambda i, m: (m[0, i], 0, 0)),
            pl.BlockSpec((1, F, D), lambda i, m: (m[0, i], 0, 0)),
        ],
        out_specs=pl.BlockSpec((BLK, D), lambda i, m: (m[1, i], 0)),
    )
    return pl.pallas_call(
        _ffn_body,
        grid_spec=grid_spec,
        out_shape=jax.ShapeDtypeStruct(x_sorted.shape, jnp.float32),
    )(meta, x_sorted, w1, w3, w2)


def _sc_combine(ys, pos2, w0f, w1f, T):
    npad, D = ys.shape
    per = T // NW
    GC = 16
    mesh = plsc.VectorSubcoreMesh(core_axis_name="c", subcore_axis_name="s")

    @functools.partial(
        pl.kernel, mesh=mesh,
        out_type=jax.ShapeDtypeStruct((T, D), jnp.float32),
        scratch_types=[
            pltpu.VMEM((per,), jnp.int32),
            pltpu.VMEM((per,), jnp.int32),
            pltpu.VMEM((per * 16,), jnp.float32),
            pltpu.VMEM((per * 16,), jnp.float32),
            pltpu.VMEM((GC, D), jnp.float32),
            pltpu.VMEM((GC, D), jnp.float32),
            pltpu.VMEM((GC, D), jnp.float32),
            pltpu.SemaphoreType.DMA,
            pltpu.SemaphoreType.DMA,
        ],
    )
    def k(ys_hbm, pos_hbm, w0_hbm, w1_hbm, out_hbm,
          idx0_v, idx1_v, w0_v, w1_v, rows0_v, rows1_v, out_v, sem0, sem1):
        wid = lax.axis_index("s") * 2 + lax.axis_index("c")
        tbase = wid * per
        pltpu.sync_copy(pos_hbm.at[0, wid], idx0_v)
        pltpu.sync_copy(pos_hbm.at[1, wid], idx1_v)
        pltpu.sync_copy(w0_hbm.at[pl.ds(tbase * 16, per * 16)], w0_v)
        pltpu.sync_copy(w1_hbm.at[pl.ds(tbase * 16, per * 16)], w1_v)

        def body(c, carry):
            cp0 = pltpu.async_copy(
                ys_hbm.at[idx0_v.at[pl.ds(c * GC, GC)]], rows0_v, sem0)
            cp1 = pltpu.async_copy(
                ys_hbm.at[idx1_v.at[pl.ds(c * GC, GC)]], rows1_v, sem1)
            cp0.wait()
            cp1.wait()
            for t in range(GC):
                wv0 = w0_v[pl.ds(c * (GC * 16) + t * 16, 16)]
                wv1 = w1_v[pl.ds(c * (GC * 16) + t * 16, 16)]
                for j in range(D // 16):
                    sl = pl.ds(j * 16, 16)
                    out_v[t, sl] = (wv0 * rows0_v[t, sl] +
                                    wv1 * rows1_v[t, sl])
            pltpu.sync_copy(out_v, out_hbm.at[pl.ds(tbase + c * GC, GC)])
            return carry

        lax.fori_loop(0, per // GC, body, 0)

    return k(ys, pos2, w0f, w1f)


def kernel(x, pressure, temperature, Wg, w1, w3, w2):
    T, D = x.shape
    E = Wg.shape[1]
    TK = T * NK
    NBLK = TK // BLK + NE
    NPAD = NBLK * BLK

    invt = 1.0 / jnp.clip(temperature, 0.3, 3.0)
    bias = 0.1 * jnp.clip(pressure, -1.0, 1.0)
    invt_pad = jnp.zeros((1, 128), jnp.float32).at[0, :E].set(invt)
    bias_pad = jnp.full((1, 128), NEG, jnp.float32).at[0, :E].set(bias)
    wg_pad = jnp.zeros((D, 128), jnp.float32).at[:, :E].set(Wg)

    w0b, w1b, pos, meta = _route_bin(x, wg_pad, invt_pad, bias_pad)

    x_sorted = _sc_scatter_x(x, pos.reshape(NW, TK // NW // 64, 64), NPAD)

    y_sorted = _ffn(x_sorted, w1, w3, w2, meta, NBLK)

    return _sc_combine(y_sorted, pos.reshape(NK, NW, T // NW),
                       w0b.reshape(T * 16), w1b.reshape(T * 16), T)


# speedup vs baseline: 1.3504x; 1.0401x over previous
import functools

import jax
import jax.numpy as jnp
from jax import lax
from jax.experimental import pallas as pl
from jax.experimental.pallas import tpu as pltpu
from jax.experimental.pallas import tpu_sc as plsc

NE = 8
NK = 2
BLK = 256
NEG = -1e30
NW = 32


def _route_bin_body(x_ref, wg_ref, invt_ref, bias_ref,
                    w0_ref, w1_ref, pos_ref, meta_ref):
    f32 = jnp.float32
    T = x_ref.shape[0]
    TK = T * NK
    C = 128

    g = jnp.dot(x_ref[...], wg_ref[...], preferred_element_type=f32)
    g = g * invt_ref[...] + bias_ref[...]
    i1 = jnp.argmax(g, axis=1).astype(jnp.int32)
    l1 = jnp.max(g, axis=1)
    lanes = lax.broadcasted_iota(jnp.int32, g.shape, 1)
    g2 = jnp.where(lanes == i1[:, None], NEG, g)
    i2 = jnp.argmax(g2, axis=1).astype(jnp.int32)
    l2 = jnp.max(g2, axis=1)
    wa = 1.0 / (1.0 + jnp.exp(l2 - l1))
    w0_ref[...] = jnp.broadcast_to(wa[:, None], (T, 16))
    w1_ref[...] = jnp.broadcast_to((1.0 - wa)[:, None], (T, 16))

    M = jnp.concatenate([(lanes == i1[:, None]).astype(f32),
                         (lanes == i2[:, None]).astype(f32)], axis=0)

    li = lax.broadcasted_iota(jnp.int32, (C, C), 0)
    lj = lax.broadcasted_iota(jnp.int32, (C, C), 1)
    ltri = (lj <= li).astype(jnp.bfloat16)
    ones = jnp.ones((C, C), jnp.bfloat16)

    counts = jnp.sum(M, axis=0, keepdims=True)
    nblk = jnp.floor((counts + (BLK - 1)) * (1.0 / BLK))
    utri = (li < lj).astype(jnp.bfloat16)
    bstart = jnp.dot(nblk.astype(jnp.bfloat16), utri,
                     preferred_element_type=f32)
    start = bstart * BLK
    used = jnp.sum(nblk, axis=1, keepdims=True)

    run = jnp.zeros((1, C), f32)
    for c in range(TK // C):
        Mc = M[c * C:(c + 1) * C, :]
        cumc = jnp.dot(ltri, Mc.astype(jnp.bfloat16),
                       preferred_element_type=f32) + run
        posc = jnp.sum(Mc * (cumc - 1.0 + start), axis=1, keepdims=True)
        pos_ref[c * C:(c + 1) * C, :] = posc.astype(jnp.int32)
        run = run + jnp.sum(Mc, axis=0, keepdims=True)

    eq = (li == lj).astype(f32)
    bstart_s = jnp.dot((eq * bstart).astype(jnp.bfloat16), ones,
                       preferred_element_type=f32)
    nblk_s = jnp.dot((eq * nblk).astype(jnp.bfloat16), ones,
                     preferred_element_type=f32)
    bidx = lj.astype(f32)
    inr = (bidx >= bstart_s) & (bidx < bstart_s + nblk_s)
    bexp_real = jnp.sum(jnp.where(inr, li.astype(f32), 0.0), axis=0,
                        keepdims=True)
    lane1 = lanes[:1, :].astype(f32)
    elast = jnp.max(jnp.where(counts > 0, lane1, -1.0), axis=1,
                    keepdims=True)
    bvalid = lane1 < used
    bexp = jnp.where(bvalid, bexp_real, elast)
    xi = jnp.where(bvalid, lane1, used - 1.0)
    meta = jnp.concatenate(
        [bexp, xi, bvalid.astype(f32), jnp.zeros((5, C), f32)], axis=0)
    meta_ref[...] = meta.astype(jnp.int32)


def _route_bin(x, wg_pad, invt_pad, bias_pad):
    T, D = x.shape
    TK = T * NK
    return pl.pallas_call(
        _route_bin_body,
        in_specs=[
            pl.BlockSpec((T, D), lambda: (0, 0)),
            pl.BlockSpec((D, 128), lambda: (0, 0)),
            pl.BlockSpec((1, 128), lambda: (0, 0)),
            pl.BlockSpec((1, 128), lambda: (0, 0)),
        ],
        out_specs=[
            pl.BlockSpec((T, 16), lambda: (0, 0)),
            pl.BlockSpec((T, 16), lambda: (0, 0)),
            pl.BlockSpec((TK, 1), lambda: (0, 0)),
            pl.BlockSpec((8, 128), lambda: (0, 0)),
        ],
        out_shape=[
            jax.ShapeDtypeStruct((T, 16), jnp.float32),
            jax.ShapeDtypeStruct((T, 16), jnp.float32),
            jax.ShapeDtypeStruct((TK, 1), jnp.int32),
            jax.ShapeDtypeStruct((8, 128), jnp.int32),
        ],
    )(x, wg_pad, invt_pad, bias_pad)


def _sc_scatter_x(x, pos3, npad):
    T, D = x.shape
    jc = pos3.shape[1]
    cb = pos3.shape[2]
    mesh = plsc.VectorSubcoreMesh(core_axis_name="c", subcore_axis_name="s")

    @functools.partial(
        pl.kernel, mesh=mesh,
        out_type=jax.ShapeDtypeStruct((npad, D), jnp.float32),
        scratch_types=[
            pltpu.VMEM((jc, cb), jnp.int32),
            pltpu.VMEM((cb, D), jnp.float32),
            pltpu.VMEM((cb, D), jnp.float32),
            pltpu.SemaphoreType.DMA,
            pltpu.SemaphoreType.DMA,
            pltpu.SemaphoreType.DMA,
            pltpu.SemaphoreType.DMA,
        ],
    )
    def k(x_hbm, pos_hbm, xs_hbm, pos_v, rows_a, rows_b, sem_a, sem_b,
          sem_sa, sem_sb):
        wid = lax.axis_index("s") * 2 + lax.axis_index("c")
        pltpu.sync_copy(pos_hbm.at[wid], pos_v)
        tok_base = wid * (jc * cb) - jnp.where(wid >= NW // 2, T, 0)
        bufs = (rows_a, rows_b)
        rsems = (sem_a, sem_b)
        ssems = (sem_sa, sem_sb)
        reads = {}
        scats = {}
        for c in range(min(2, jc)):
            reads[c] = pltpu.async_copy(
                x_hbm.at[pl.ds(tok_base + c * cb, cb)], bufs[c % 2],
                rsems[c % 2])
        for c in range(jc):
            reads[c].wait()
            scats[c] = pltpu.async_copy(
                bufs[c % 2], xs_hbm.at[pos_v.at[c]], ssems[c % 2])
            if c + 2 < jc:
                scats[c].wait()
                reads[c + 2] = pltpu.async_copy(
                    x_hbm.at[pl.ds(tok_base + (c + 2) * cb, cb)],
                    bufs[c % 2], rsems[c % 2])
        for c in range(max(0, jc - 2), jc):
            scats[c].wait()

    return k(x, pos3)


def _ffn_body(meta_ref, xs_ref, w1_ref, w3_ref, w2_ref, ys_ref):
    i = pl.program_id(0)

    @pl.when(meta_ref[2, i] != 0)
    def _():
        xb = xs_ref[...].astype(jnp.bfloat16)
        a = jnp.dot(xb, w1_ref[0].astype(jnp.bfloat16),
                    preferred_element_type=jnp.float32)
        b = jnp.dot(xb, w3_ref[0].astype(jnp.bfloat16),
                    preferred_element_type=jnp.float32)
        h = (a * jax.nn.sigmoid(a) * b).astype(jnp.bfloat16)
        ys_ref[...] = jnp.dot(h, w2_ref[0].astype(jnp.bfloat16),
                              preferred_element_type=jnp.float32)


def _ffn(x_sorted, w1, w3, w2, meta, nblk):
    D = x_sorted.shape[1]
    F = w1.shape[2]
    grid_spec = pltpu.PrefetchScalarGridSpec(
        num_scalar_prefetch=1,
        grid=(nblk,),
        in_specs=[
            pl.BlockSpec((BLK, D), lambda i, m: (m[1, i], 0)),
            pl.BlockSpec((1, D, F), lambda i, m: (m[0, i], 0, 0)),
            pl.BlockSpec((1, D, F), lambda i, m: (m[0, i], 0, 0)),
            pl.BlockSpec((1, F, D), lambda i, m: (m[0, i], 0, 0)),
        ],
        out_specs=pl.BlockSpec((BLK, D), lambda i, m: (m[1, i], 0)),
    )
    return pl.pallas_call(
        _ffn_body,
        grid_spec=grid_spec,
        out_shape=jax.ShapeDtypeStruct(x_sorted.shape, jnp.float32),
    )(meta, x_sorted, w1, w3, w2)


def _sc_combine(ys, pos2, w0f, w1f, T):
    npad, D = ys.shape
    per = T // NW
    GC = 16
    mesh = plsc.VectorSubcoreMesh(core_axis_name="c", subcore_axis_name="s")

    @functools.partial(
        pl.kernel, mesh=mesh,
        out_type=jax.ShapeDtypeStruct((T, D), jnp.float32),
        scratch_types=[
            pltpu.VMEM((per,), jnp.int32),
            pltpu.VMEM((per,), jnp.int32),
            pltpu.VMEM((per * 16,), jnp.float32),
            pltpu.VMEM((per * 16,), jnp.float32),
            pltpu.VMEM((GC, D), jnp.float32),
            pltpu.VMEM((GC, D), jnp.float32),
            pltpu.VMEM((GC, D), jnp.float32),
            pltpu.VMEM((GC, D), jnp.float32),
            pltpu.VMEM((GC, D), jnp.float32),
            pltpu.VMEM((GC, D), jnp.float32),
            pltpu.SemaphoreType.DMA,
            pltpu.SemaphoreType.DMA,
            pltpu.SemaphoreType.DMA,
            pltpu.SemaphoreType.DMA,
            pltpu.SemaphoreType.DMA,
            pltpu.SemaphoreType.DMA,
        ],
    )
    def k(ys_hbm, pos_hbm, w0_hbm, w1_hbm, out_hbm,
          idx0_v, idx1_v, w0_v, w1_v, r0a, r1a, r0b, r1b, oa, ob,
          s0a, s1a, s0b, s1b, soa, sob):
        wid = lax.axis_index("s") * 2 + lax.axis_index("c")
        tbase = wid * per
        pltpu.sync_copy(pos_hbm.at[0, wid], idx0_v)
        pltpu.sync_copy(pos_hbm.at[1, wid], idx1_v)
        pltpu.sync_copy(w0_hbm.at[pl.ds(tbase * 16, per * 16)], w0_v)
        pltpu.sync_copy(w1_hbm.at[pl.ds(tbase * 16, per * 16)], w1_v)

        def fire(c, r0, r1, se0, se1):
            d0 = pltpu.async_copy(
                ys_hbm.at[idx0_v.at[pl.ds(c * GC, GC)]], r0, se0)
            d1 = pltpu.async_copy(
                ys_hbm.at[idx1_v.at[pl.ds(c * GC, GC)]], r1, se1)
            return d0, d1

        def compute(c, r0, r1, o):
            for t in range(GC):
                wv0 = w0_v[pl.ds(c * (GC * 16) + t * 16, 16)]
                wv1 = w1_v[pl.ds(c * (GC * 16) + t * 16, 16)]
                for j in range(D // 16):
                    sl = pl.ds(j * 16, 16)
                    o[t, sl] = wv0 * r0[t, sl] + wv1 * r1[t, sl]

        fire(0, r0a, r1a, s0a, s1a)

        def body(i, carry):
            ca = 2 * i
            cb2 = 2 * i + 1
            db0, db1 = fire(cb2, r0b, r1b, s0b, s1b)
            pltpu.make_async_copy(ys_hbm.at[idx0_v.at[pl.ds(0, GC)]],
                                  r0a, s0a).wait()
            pltpu.make_async_copy(ys_hbm.at[idx0_v.at[pl.ds(0, GC)]],
                                  r1a, s1a).wait()

            @pl.when(i > 0)
            def _():
                pltpu.make_async_copy(oa, out_hbm.at[pl.ds(tbase, GC)],
                                      soa).wait()

            compute(ca, r0a, r1a, oa)
            pltpu.async_copy(oa, out_hbm.at[pl.ds(tbase + ca * GC, GC)], soa)

            @pl.when(i == 0)
            def _():
                fire(2, r0a, r1a, s0a, s1a)

            db0.wait()
            db1.wait()

            @pl.when(i > 0)
            def _():
                pltpu.make_async_copy(ob, out_hbm.at[pl.ds(tbase, GC)],
                                      sob).wait()

            compute(cb2, r0b, r1b, ob)
            pltpu.async_copy(ob, out_hbm.at[pl.ds(tbase + cb2 * GC, GC)], sob)
            return carry

        lax.fori_loop(0, per // GC // 2, body, 0)
        pltpu.make_async_copy(oa, out_hbm.at[pl.ds(tbase, GC)], soa).wait()
        pltpu.make_async_copy(ob, out_hbm.at[pl.ds(tbase, GC)], sob).wait()

    return k(ys, pos2, w0f, w1f)


def kernel(x, pressure, temperature, Wg, w1, w3, w2):
    T, D = x.shape
    E = Wg.shape[1]
    TK = T * NK
    NBLK = TK // BLK + NE
    NPAD = NBLK * BLK

    invt = 1.0 / jnp.clip(temperature, 0.3, 3.0)
    bias = 0.1 * jnp.clip(pressure, -1.0, 1.0)
    invt_pad = jnp.zeros((1, 128), jnp.float32).at[0, :E].set(invt)
    bias_pad = jnp.full((1, 128), NEG, jnp.float32).at[0, :E].set(bias)
    wg_pad = jnp.zeros((D, 128), jnp.float32).at[:, :E].set(Wg)

    w0b, w1b, pos, meta = _route_bin(x, wg_pad, invt_pad, bias_pad)

    x_sorted = _sc_scatter_x(x, pos.reshape(NW, TK // NW // 32, 32), NPAD)

    y_sorted = _ffn(x_sorted, w1, w3, w2, meta, NBLK)

    return _sc_combine(y_sorted, pos.reshape(NK, NW, T // NW),
                       w0b.reshape(T * 16), w1b.reshape(T * 16), T)


# speedup vs baseline: 1.3584x; 1.0059x over previous
import functools

import jax
import jax.numpy as jnp
from jax import lax
from jax.experimental import pallas as pl
from jax.experimental.pallas import tpu as pltpu
from jax.experimental.pallas import tpu_sc as plsc

NE = 8
NK = 2
BLK = 256
NEG = -1e30
NW = 32


def _route_bin_body(x_ref, wg_ref, invt_ref, bias_ref,
                    w0_ref, w1_ref, pos_ref, meta_ref):
    f32 = jnp.float32
    T = x_ref.shape[0]
    TK = T * NK
    C = 128

    g = jnp.dot(x_ref[...], wg_ref[...], preferred_element_type=f32)
    g = g * invt_ref[...] + bias_ref[...]
    i1 = jnp.argmax(g, axis=1).astype(jnp.int32)
    l1 = jnp.max(g, axis=1)
    lanes = lax.broadcasted_iota(jnp.int32, g.shape, 1)
    g2 = jnp.where(lanes == i1[:, None], NEG, g)
    i2 = jnp.argmax(g2, axis=1).astype(jnp.int32)
    l2 = jnp.max(g2, axis=1)
    wa = 1.0 / (1.0 + jnp.exp(l2 - l1))
    w0_ref[...] = jnp.broadcast_to(wa[:, None], (T, 16))
    w1_ref[...] = jnp.broadcast_to((1.0 - wa)[:, None], (T, 16))

    M = jnp.concatenate([(lanes == i1[:, None]).astype(f32),
                         (lanes == i2[:, None]).astype(f32)], axis=0)

    li = lax.broadcasted_iota(jnp.int32, (C, C), 0)
    lj = lax.broadcasted_iota(jnp.int32, (C, C), 1)
    ltri = (lj <= li).astype(jnp.bfloat16)
    ones = jnp.ones((C, C), jnp.bfloat16)

    counts = jnp.sum(M, axis=0, keepdims=True)
    nblk = jnp.floor((counts + (BLK - 1)) * (1.0 / BLK))
    utri = (li < lj).astype(jnp.bfloat16)
    bstart = jnp.dot(nblk.astype(jnp.bfloat16), utri,
                     preferred_element_type=f32)
    start = bstart * BLK
    used = jnp.sum(nblk, axis=1, keepdims=True)

    run = jnp.zeros((1, C), f32)
    for c in range(TK // C):
        Mc = M[c * C:(c + 1) * C, :]
        cumc = jnp.dot(ltri, Mc.astype(jnp.bfloat16),
                       preferred_element_type=f32) + run
        posc = jnp.sum(Mc * (cumc - 1.0 + start), axis=1, keepdims=True)
        pos_ref[c * C:(c + 1) * C, :] = posc.astype(jnp.int32)
        run = run + jnp.sum(Mc, axis=0, keepdims=True)

    eq = (li == lj).astype(f32)
    bstart_s = jnp.dot((eq * bstart).astype(jnp.bfloat16), ones,
                       preferred_element_type=f32)
    nblk_s = jnp.dot((eq * nblk).astype(jnp.bfloat16), ones,
                     preferred_element_type=f32)
    bidx = lj.astype(f32)
    inr = (bidx >= bstart_s) & (bidx < bstart_s + nblk_s)
    bexp_real = jnp.sum(jnp.where(inr, li.astype(f32), 0.0), axis=0,
                        keepdims=True)
    lane1 = lanes[:1, :].astype(f32)
    elast = jnp.max(jnp.where(counts > 0, lane1, -1.0), axis=1,
                    keepdims=True)
    bvalid = lane1 < used
    bexp = jnp.where(bvalid, bexp_real, elast)
    xi = jnp.where(bvalid, lane1, used - 1.0)
    meta = jnp.concatenate(
        [bexp, xi, bvalid.astype(f32), jnp.zeros((5, C), f32)], axis=0)
    meta_ref[...] = meta.astype(jnp.int32)


def _route_bin(x, wg_pad, invt_pad, bias_pad):
    T, D = x.shape
    TK = T * NK
    return pl.pallas_call(
        _route_bin_body,
        in_specs=[
            pl.BlockSpec((T, D), lambda: (0, 0)),
            pl.BlockSpec((D, 128), lambda: (0, 0)),
            pl.BlockSpec((1, 128), lambda: (0, 0)),
            pl.BlockSpec((1, 128), lambda: (0, 0)),
        ],
        out_specs=[
            pl.BlockSpec((T, 16), lambda: (0, 0)),
            pl.BlockSpec((T, 16), lambda: (0, 0)),
            pl.BlockSpec((TK, 1), lambda: (0, 0)),
            pl.BlockSpec((8, 128), lambda: (0, 0)),
        ],
        out_shape=[
            jax.ShapeDtypeStruct((T, 16), jnp.float32),
            jax.ShapeDtypeStruct((T, 16), jnp.float32),
            jax.ShapeDtypeStruct((TK, 1), jnp.int32),
            jax.ShapeDtypeStruct((8, 128), jnp.int32),
        ],
    )(x, wg_pad, invt_pad, bias_pad)


def _sc_scatter_x(x, pos3, npad):
    T, D = x.shape
    jc = pos3.shape[1]
    cb = pos3.shape[2]
    mesh = plsc.VectorSubcoreMesh(core_axis_name="c", subcore_axis_name="s")

    @functools.partial(
        pl.kernel, mesh=mesh,
        out_type=jax.ShapeDtypeStruct((npad, D), jnp.float32),
        scratch_types=[
            pltpu.VMEM((jc, cb), jnp.int32),
            pltpu.VMEM((cb, D), jnp.float32),
            pltpu.VMEM((cb, D), jnp.float32),
            pltpu.SemaphoreType.DMA,
            pltpu.SemaphoreType.DMA,
            pltpu.SemaphoreType.DMA,
            pltpu.SemaphoreType.DMA,
        ],
    )
    def k(x_hbm, pos_hbm, xs_hbm, pos_v, rows_a, rows_b, sem_a, sem_b,
          sem_sa, sem_sb):
        wid = lax.axis_index("s") * 2 + lax.axis_index("c")
        pltpu.sync_copy(pos_hbm.at[wid], pos_v)
        tok_base = wid * (jc * cb) - jnp.where(wid >= NW // 2, T, 0)
        bufs = (rows_a, rows_b)
        rsems = (sem_a, sem_b)
        ssems = (sem_sa, sem_sb)
        reads = {}
        scats = {}
        for c in range(min(2, jc)):
            reads[c] = pltpu.async_copy(
                x_hbm.at[pl.ds(tok_base + c * cb, cb)], bufs[c % 2],
                rsems[c % 2])
        for c in range(jc):
            reads[c].wait()
            scats[c] = pltpu.async_copy(
                bufs[c % 2], xs_hbm.at[pos_v.at[c]], ssems[c % 2])
            if c + 2 < jc:
                scats[c].wait()
                reads[c + 2] = pltpu.async_copy(
                    x_hbm.at[pl.ds(tok_base + (c + 2) * cb, cb)],
                    bufs[c % 2], rsems[c % 2])
        for c in range(max(0, jc - 2), jc):
            scats[c].wait()

    return k(x, pos3)


def _ffn_body(meta_ref, xs_ref, w1_ref, w3_ref, w2_ref, ys_ref):
    i = pl.program_id(0)

    @pl.when(meta_ref[2, i] != 0)
    def _():
        xb = xs_ref[...].astype(jnp.bfloat16)
        a = jnp.dot(xb, w1_ref[0].astype(jnp.bfloat16),
                    preferred_element_type=jnp.float32)
        b = jnp.dot(xb, w3_ref[0].astype(jnp.bfloat16),
                    preferred_element_type=jnp.float32)
        h = (a * jax.nn.sigmoid(a) * b).astype(jnp.bfloat16)
        ys_ref[...] = jnp.dot(h, w2_ref[0].astype(jnp.bfloat16),
                              preferred_element_type=jnp.float32)


def _ffn(x_sorted, w1, w3, w2, meta, nblk):
    D = x_sorted.shape[1]
    F = w1.shape[2]
    grid_spec = pltpu.PrefetchScalarGridSpec(
        num_scalar_prefetch=1,
        grid=(nblk,),
        in_specs=[
            pl.BlockSpec((BLK, D), lambda i, m: (m[1, i], 0)),
            pl.BlockSpec((1, D, F), lambda i, m: (m[0, i], 0, 0)),
            pl.BlockSpec((1, D, F), lambda i, m: (m[0, i], 0, 0)),
            pl.BlockSpec((1, F, D), lambda i, m: (m[0, i], 0, 0)),
        ],
        out_specs=pl.BlockSpec((BLK, D), lambda i, m: (m[1, i], 0)),
    )
    return pl.pallas_call(
        _ffn_body,
        grid_spec=grid_spec,
        out_shape=jax.ShapeDtypeStruct(x_sorted.shape, jnp.float32),
    )(meta, x_sorted, w1, w3, w2)


def _sc_combine(ys, pos2, w0f, w1f, T):
    npad, D = ys.shape
    per = T // NW
    GC = 16
    mesh = plsc.VectorSubcoreMesh(core_axis_name="c", subcore_axis_name="s")

    @functools.partial(
        pl.kernel, mesh=mesh,
        out_type=jax.ShapeDtypeStruct((T, D), jnp.float32),
        scratch_types=[
            pltpu.VMEM((per,), jnp.int32),
            pltpu.VMEM((per,), jnp.int32),
            pltpu.VMEM((per * 16,), jnp.float32),
            pltpu.VMEM((per * 16,), jnp.float32),
            pltpu.VMEM((GC, D), jnp.float32),
            pltpu.VMEM((GC, D), jnp.float32),
            pltpu.VMEM((GC, D), jnp.float32),
            pltpu.SemaphoreType.DMA,
            pltpu.SemaphoreType.DMA,
        ],
    )
    def k(ys_hbm, pos_hbm, w0_hbm, w1_hbm, out_hbm,
          idx0_v, idx1_v, w0_v, w1_v, rows0_v, rows1_v, out_v, sem0, sem1):
        wid = lax.axis_index("s") * 2 + lax.axis_index("c")
        tbase = wid * per
        pltpu.sync_copy(pos_hbm.at[0, wid], idx0_v)
        pltpu.sync_copy(pos_hbm.at[1, wid], idx1_v)
        pltpu.sync_copy(w0_hbm.at[pl.ds(tbase * 16, per * 16)], w0_v)
        pltpu.sync_copy(w1_hbm.at[pl.ds(tbase * 16, per * 16)], w1_v)

        def body(c, carry):
            cp0 = pltpu.async_copy(
                ys_hbm.at[idx0_v.at[pl.ds(c * GC, GC)]], rows0_v, sem0)
            cp1 = pltpu.async_copy(
                ys_hbm.at[idx1_v.at[pl.ds(c * GC, GC)]], rows1_v, sem1)
            cp0.wait()
            cp1.wait()
            for t in range(GC):
                wv0 = w0_v[pl.ds(c * (GC * 16) + t * 16, 16)]
                wv1 = w1_v[pl.ds(c * (GC * 16) + t * 16, 16)]
                for j in range(D // 16):
                    sl = pl.ds(j * 16, 16)
                    out_v[t, sl] = (wv0 * rows0_v[t, sl] +
                                    wv1 * rows1_v[t, sl])
            pltpu.sync_copy(out_v, out_hbm.at[pl.ds(tbase + c * GC, GC)])
            return carry

        lax.fori_loop(0, per // GC, body, 0)

    return k(ys, pos2, w0f, w1f)


def kernel(x, pressure, temperature, Wg, w1, w3, w2):
    T, D = x.shape
    E = Wg.shape[1]
    TK = T * NK
    NBLK = TK // BLK + NE
    NPAD = NBLK * BLK

    invt = 1.0 / jnp.clip(temperature, 0.3, 3.0)
    bias = 0.1 * jnp.clip(pressure, -1.0, 1.0)
    invt_pad = jnp.zeros((1, 128), jnp.float32).at[0, :E].set(invt)
    bias_pad = jnp.full((1, 128), NEG, jnp.float32).at[0, :E].set(bias)
    wg_pad = jnp.zeros((D, 128), jnp.float32).at[:, :E].set(Wg)

    w0b, w1b, pos, meta = _route_bin(x, wg_pad, invt_pad, bias_pad)

    x_sorted = _sc_scatter_x(x, pos.reshape(NW, TK // NW // 32, 32), NPAD)

    y_sorted = _ffn(x_sorted, w1, w3, w2, meta, NBLK)

    return _sc_combine(y_sorted, pos.reshape(NK, NW, T // NW),
                       w0b.reshape(T * 16), w1b.reshape(T * 16), T)


# speedup vs baseline: 1.3686x; 1.0075x over previous
import functools

import jax
import jax.numpy as jnp
from jax import lax
from jax.experimental import pallas as pl
from jax.experimental.pallas import tpu as pltpu
from jax.experimental.pallas import tpu_sc as plsc

NE = 8
NK = 2
BLK = 256
NEG = -1e30
NW = 32


def _route_bin_body(x_ref, wg_ref, invt_ref, bias_ref,
                    w0_ref, w1_ref, pos_ref, meta_ref):
    f32 = jnp.float32
    T = x_ref.shape[0]
    TK = T * NK
    C = 128

    g = jnp.dot(x_ref[...], wg_ref[...], preferred_element_type=f32)
    g = g * invt_ref[...] + bias_ref[...]
    i1 = jnp.argmax(g, axis=1).astype(jnp.int32)
    l1 = jnp.max(g, axis=1)
    lanes = lax.broadcasted_iota(jnp.int32, g.shape, 1)
    g2 = jnp.where(lanes == i1[:, None], NEG, g)
    i2 = jnp.argmax(g2, axis=1).astype(jnp.int32)
    l2 = jnp.max(g2, axis=1)
    wa = 1.0 / (1.0 + jnp.exp(l2 - l1))
    w0_ref[...] = jnp.broadcast_to(wa[:, None], (T, 16))
    w1_ref[...] = jnp.broadcast_to((1.0 - wa)[:, None], (T, 16))

    M = jnp.concatenate([(lanes == i1[:, None]).astype(f32),
                         (lanes == i2[:, None]).astype(f32)], axis=0)

    li = lax.broadcasted_iota(jnp.int32, (C, C), 0)
    lj = lax.broadcasted_iota(jnp.int32, (C, C), 1)
    ltri = (lj <= li).astype(jnp.bfloat16)
    ones = jnp.ones((C, C), jnp.bfloat16)

    counts = jnp.sum(M, axis=0, keepdims=True)
    nblk = jnp.floor((counts + (BLK - 1)) * (1.0 / BLK))
    utri = (li < lj).astype(jnp.bfloat16)
    bstart = jnp.dot(nblk.astype(jnp.bfloat16), utri,
                     preferred_element_type=f32)
    start = bstart * BLK
    used = jnp.sum(nblk, axis=1, keepdims=True)

    run = jnp.zeros((1, C), f32)
    for c in range(TK // C):
        Mc = M[c * C:(c + 1) * C, :]
        cumc = jnp.dot(ltri, Mc.astype(jnp.bfloat16),
                       preferred_element_type=f32) + run
        posc = jnp.sum(Mc * (cumc - 1.0 + start), axis=1, keepdims=True)
        pos_ref[c * C:(c + 1) * C, :] = posc.astype(jnp.int32)
        run = run + jnp.sum(Mc, axis=0, keepdims=True)

    eq = (li == lj).astype(f32)
    bstart_s = jnp.dot((eq * bstart).astype(jnp.bfloat16), ones,
                       preferred_element_type=f32)
    nblk_s = jnp.dot((eq * nblk).astype(jnp.bfloat16), ones,
                     preferred_element_type=f32)
    bidx = lj.astype(f32)
    inr = (bidx >= bstart_s) & (bidx < bstart_s + nblk_s)
    bexp_real = jnp.sum(jnp.where(inr, li.astype(f32), 0.0), axis=0,
                        keepdims=True)
    lane1 = lanes[:1, :].astype(f32)
    elast = jnp.max(jnp.where(counts > 0, lane1, -1.0), axis=1,
                    keepdims=True)
    bvalid = lane1 < used
    bexp = jnp.where(bvalid, bexp_real, elast)
    xi = jnp.where(bvalid, lane1, used - 1.0)
    meta = jnp.concatenate(
        [bexp, xi, bvalid.astype(f32), jnp.zeros((5, C), f32)], axis=0)
    meta_ref[...] = meta.astype(jnp.int32)


def _route_bin(x, wg_pad, invt_pad, bias_pad):
    T, D = x.shape
    TK = T * NK
    return pl.pallas_call(
        _route_bin_body,
        in_specs=[
            pl.BlockSpec((T, D), lambda: (0, 0)),
            pl.BlockSpec((D, 128), lambda: (0, 0)),
            pl.BlockSpec((1, 128), lambda: (0, 0)),
            pl.BlockSpec((1, 128), lambda: (0, 0)),
        ],
        out_specs=[
            pl.BlockSpec((T, 16), lambda: (0, 0)),
            pl.BlockSpec((T, 16), lambda: (0, 0)),
            pl.BlockSpec((TK, 1), lambda: (0, 0)),
            pl.BlockSpec((8, 128), lambda: (0, 0)),
        ],
        out_shape=[
            jax.ShapeDtypeStruct((T, 16), jnp.float32),
            jax.ShapeDtypeStruct((T, 16), jnp.float32),
            jax.ShapeDtypeStruct((TK, 1), jnp.int32),
            jax.ShapeDtypeStruct((8, 128), jnp.int32),
        ],
    )(x, wg_pad, invt_pad, bias_pad)


def _sc_scatter_x(x, pos3, npad):
    T, D = x.shape
    jc = pos3.shape[1]
    cb = pos3.shape[2]
    mesh = plsc.VectorSubcoreMesh(core_axis_name="c", subcore_axis_name="s")

    @functools.partial(
        pl.kernel, mesh=mesh,
        out_type=jax.ShapeDtypeStruct((npad, D), jnp.float32),
        scratch_types=[
            pltpu.VMEM((jc, cb), jnp.int32),
            pltpu.VMEM((cb, D), jnp.float32),
            pltpu.SemaphoreType.DMA,
        ],
    )
    def k(x_hbm, pos_hbm, xs_hbm, pos_v, rows_v, sem):
        wid = lax.axis_index("s") * 2 + lax.axis_index("c")
        pltpu.sync_copy(pos_hbm.at[wid], pos_v)
        tok_base = wid * (jc * cb) - jnp.where(wid >= NW // 2, T, 0)
        for c in range(jc):
            pltpu.sync_copy(x_hbm.at[pl.ds(tok_base + c * cb, cb)], rows_v)
            pltpu.async_copy(rows_v, xs_hbm.at[pos_v.at[c]], sem).wait()

    return k(x, pos3)


def _ffn_body(meta_ref, xs_ref, w1_ref, w3_ref, w2_ref, ys_ref):
    i = pl.program_id(0)

    @pl.when(meta_ref[2, i] != 0)
    def _():
        xb = xs_ref[...].astype(jnp.bfloat16)
        a = jnp.dot(xb, w1_ref[0].astype(jnp.bfloat16),
                    preferred_element_type=jnp.float32)
        b = jnp.dot(xb, w3_ref[0].astype(jnp.bfloat16),
                    preferred_element_type=jnp.float32)
        h = (a * jax.nn.sigmoid(a) * b).astype(jnp.bfloat16)
        ys_ref[...] = jnp.dot(h, w2_ref[0].astype(jnp.bfloat16),
                              preferred_element_type=jnp.float32)


def _ffn(x_sorted, w1, w3, w2, meta, nblk):
    D = x_sorted.shape[1]
    F = w1.shape[2]
    grid_spec = pltpu.PrefetchScalarGridSpec(
        num_scalar_prefetch=1,
        grid=(nblk,),
        in_specs=[
            pl.BlockSpec((BLK, D), lambda i, m: (m[1, i], 0)),
            pl.BlockSpec((1, D, F), lambda i, m: (m[0, i], 0, 0)),
            pl.BlockSpec((1, D, F), lambda i, m: (m[0, i], 0, 0)),
            pl.BlockSpec((1, F, D), lambda i, m: (m[0, i], 0, 0)),
        ],
        out_specs=pl.BlockSpec((BLK, D), lambda i, m: (m[1, i], 0)),
    )
    return pl.pallas_call(
        _ffn_body,
        grid_spec=grid_spec,
        out_shape=jax.ShapeDtypeStruct(x_sorted.shape, jnp.float32),
    )(meta, x_sorted, w1, w3, w2)


def _sc_combine(ys, pos2, w0f, w1f, T):
    npad, D = ys.shape
    per = T // NW
    GC = 16
    mesh = plsc.VectorSubcoreMesh(core_axis_name="c", subcore_axis_name="s")

    @functools.partial(
        pl.kernel, mesh=mesh,
        out_type=jax.ShapeDtypeStruct((T, D), jnp.float32),
        scratch_types=[
            pltpu.VMEM((per,), jnp.int32),
            pltpu.VMEM((per,), jnp.int32),
            pltpu.VMEM((per * 16,), jnp.float32),
            pltpu.VMEM((per * 16,), jnp.float32),
            pltpu.VMEM((GC, D), jnp.float32),
            pltpu.VMEM((GC, D), jnp.float32),
            pltpu.VMEM((GC, D), jnp.float32),
            pltpu.SemaphoreType.DMA,
            pltpu.SemaphoreType.DMA,
        ],
    )
    def k(ys_hbm, pos_hbm, w0_hbm, w1_hbm, out_hbm,
          idx0_v, idx1_v, w0_v, w1_v, rows0_v, rows1_v, out_v, sem0, sem1):
        wid = lax.axis_index("s") * 2 + lax.axis_index("c")
        tbase = wid * per
        pltpu.sync_copy(pos_hbm.at[0, wid], idx0_v)
        pltpu.sync_copy(pos_hbm.at[1, wid], idx1_v)
        pltpu.sync_copy(w0_hbm.at[pl.ds(tbase * 16, per * 16)], w0_v)
        pltpu.sync_copy(w1_hbm.at[pl.ds(tbase * 16, per * 16)], w1_v)

        def body(c, carry):
            cp0 = pltpu.async_copy(
                ys_hbm.at[idx0_v.at[pl.ds(c * GC, GC)]], rows0_v, sem0)
            cp1 = pltpu.async_copy(
                ys_hbm.at[idx1_v.at[pl.ds(c * GC, GC)]], rows1_v, sem1)
            cp0.wait()
            cp1.wait()
            for t in range(GC):
                wv0 = w0_v[pl.ds(c * (GC * 16) + t * 16, 16)]
                wv1 = w1_v[pl.ds(c * (GC * 16) + t * 16, 16)]
                for j in range(D // 16):
                    sl = pl.ds(j * 16, 16)
                    out_v[t, sl] = (wv0 * rows0_v[t, sl] +
                                    wv1 * rows1_v[t, sl])
            pltpu.sync_copy(out_v, out_hbm.at[pl.ds(tbase + c * GC, GC)])
            return carry

        lax.fori_loop(0, per // GC, body, 0)

    return k(ys, pos2, w0f, w1f)


def kernel(x, pressure, temperature, Wg, w1, w3, w2):
    T, D = x.shape
    E = Wg.shape[1]
    TK = T * NK
    NBLK = TK // BLK + NE
    NPAD = NBLK * BLK

    invt = 1.0 / jnp.clip(temperature, 0.3, 3.0)
    bias = 0.1 * jnp.clip(pressure, -1.0, 1.0)
    invt_pad = jnp.zeros((1, 128), jnp.float32).at[0, :E].set(invt)
    bias_pad = jnp.full((1, 128), NEG, jnp.float32).at[0, :E].set(bias)
    wg_pad = jnp.zeros((D, 128), jnp.float32).at[:, :E].set(Wg)

    w0b, w1b, pos, meta = _route_bin(x, wg_pad, invt_pad, bias_pad)

    x_sorted = _sc_scatter_x(x, pos.reshape(NW, TK // NW // 64, 64), NPAD)

    y_sorted = _ffn(x_sorted, w1, w3, w2, meta, NBLK)

    return _sc_combine(y_sorted, pos.reshape(NK, NW, T // NW),
                       w0b.reshape(T * 16), w1b.reshape(T * 16), T)
